# baseline (speedup 1.0000x reference)
import functools

import jax
import jax.numpy as jnp
from jax import lax
from jax.experimental import pallas as pl
from jax.experimental.pallas import tpu as pltpu
from jax.experimental.pallas import tpu_sc as plsc

VOCAB = 1000000
EMBED_DIM = 16
BATCH = 16384
HIST = 200

NC = 2
NS = 16
NW = NC * NS

B_PER_W = BATCH // NW
ROWS_PER_CHUNK = 8
CHUNKS = B_PER_W // ROWS_PER_CHUNK
CHUNK_VALS = ROWS_PER_CHUNK * HIST
VALS_PER_W = B_PER_W * HIST
SUPER_CHUNKS = 16
SUPERS = CHUNKS // SUPER_CHUNKS
VALS_PER_SUPER = SUPER_CHUNKS * CHUNK_VALS
D0, D1 = 96, 104

_RNE_MAGIC = float(2.0 ** 23)


_QBLK = 8192


def _q_body(t_ref, w_ref, o_ref):
  o_ref[...] = jnp.dot(t_ref[...], w_ref[...],
                       preferred_element_type=jnp.float32)


_q_call = pl.pallas_call(
    _q_body,
    grid=(pl.cdiv(VOCAB, _QBLK),),
    in_specs=[
        pl.BlockSpec((_QBLK, EMBED_DIM), lambda i: (i, 0)),
        pl.BlockSpec((EMBED_DIM, 1), lambda i: (0, 0)),
    ],
    out_specs=pl.BlockSpec((_QBLK, 1), lambda i: (i, 0)),
    out_shape=jax.ShapeDtypeStruct((VOCAB, 1), jnp.float32),
)


def _make_sc_kernel():
  mesh = plsc.VectorSubcoreMesh(core_axis_name="c", subcore_axis_name="s")

  @functools.partial(
      pl.kernel,
      mesh=mesh,
      compiler_params=pltpu.CompilerParams(
          needs_layout_passes=False, use_tc_tiling_on_sc=False),
      out_type=jax.ShapeDtypeStruct((BATCH,), jnp.float32),
      scratch_types=[
          pltpu.VMEM_SHARED((VOCAB,), jnp.float32),
          pltpu.VMEM((VALS_PER_SUPER,), jnp.int32),
          pltpu.VMEM((CHUNK_VALS,), jnp.float32),
          pltpu.VMEM((CHUNK_VALS,), jnp.float32),
          pltpu.VMEM((B_PER_W,), jnp.float32),
          pltpu.VMEM((EMBED_DIM,), jnp.float32),
          pltpu.VMEM((EMBED_DIM, 2 * ROWS_PER_CHUNK), jnp.float32),
          pltpu.SemaphoreType.DMA,
          pltpu.SemaphoreType.DMA,
      ],
  )
  def k(x_hbm, q_hbm, b_hbm, out_hbm,
        q_sp, idx_buf, gbuf0, gbuf1, out_v, b_v, macc, sem0, sem1):
    sid = lax.axis_index("s")
    wid = sid * NC + lax.axis_index("c")

    @pl.when(sid == 0)
    def _():
      pltpu.sync_copy(q_hbm, q_sp)

    pltpu.sync_copy(b_hbm, b_v)
    plsc.subcore_barrier()

    bv = b_v[...]
    lane = lax.iota(jnp.int32, 16)
    inv_hist = jnp.float32(1.0 / HIST)
    tail_flat = (jnp.int32(200) * lax.shift_right_logical(lane, 3)
                 + jnp.int32(192) + (lane & 7))
    tadd_rows = lane & 7
    tadd_cols = lax.shift_right_logical(lane, 3)

    def fire(c, gbuf, sem):
      for r in range(ROWS_PER_CHUNK):
        off = pl.multiple_of(c * CHUNK_VALS + r * HIST, 8)
        pltpu.async_copy(
            q_sp.at[idx_buf.at[pl.ds(off, D0)]],
            gbuf.at[pl.ds(r * HIST, D0)],
            sem)
        pltpu.async_copy(
            q_sp.at[idx_buf.at[pl.ds(off + D0, D1)]],
            gbuf.at[pl.ds(r * HIST + D0, D1)],
            sem)

    def drain(gbuf, sem):
      for r in range(ROWS_PER_CHUNK):
        pltpu.make_async_copy(
            q_sp.at[idx_buf.at[pl.ds(0, D0)]],
            gbuf.at[pl.ds(r * HIST, D0)],
            sem).wait()
        pltpu.make_async_copy(
            q_sp.at[idx_buf.at[pl.ds(D0, D1)]],
            gbuf.at[pl.ds(r * HIST + D0, D1)],
            sem).wait()

    def reduce_into(gbuf, col_base):
      for r in range(ROWS_PER_CHUNK):
        base = r * HIST
        accs = [gbuf[pl.ds(base + 16 * t, 16)] for t in range(4)]
        for t in range(4, 12):
          accs[t % 4] = accs[t % 4] + gbuf[pl.ds(base + 16 * t, 16)]
        acc = (accs[0] + accs[1]) + (accs[2] + accs[3])
        plsc.store_scatter(macc, [lane, jnp.full((16,), col_base + r,
                                                 jnp.int32)],
                           acc)
      for g in range(4):
        tails = plsc.load_gather(gbuf, [jnp.int32(400 * g) + tail_flat])
        plsc.addupdate_scatter(
            macc, [tadd_rows,
                   jnp.int32(col_base + 2 * g) + tadd_cols],
            tails)

    def epilogue(i):
      cols = [macc[d, :] for d in range(EMBED_DIM)]
      while len(cols) > 1:
        cols = [cols[i] + cols[i + 1] for i in range(0, len(cols), 2)]
      z = cols[0]
      t = z * inv_hist + bv
      p = 1.0 / (1.0 + jnp.exp(-t))
      y = p * jnp.float32(10000.0)
      y = (y + _RNE_MAGIC) - _RNE_MAGIC
      out_v[pl.ds(i * 16, 16)] = y / jnp.float32(10000.0)

    def super_body(s, carry):
      pltpu.sync_copy(
          x_hbm.at[pl.ds(wid * VALS_PER_W + s * VALS_PER_SUPER,
                         VALS_PER_SUPER)],
          idx_buf)
      fire(0, gbuf0, sem0)

      def pair_body(p, carry2):
        c0 = p * 2
        drain(gbuf0, sem0)
        fire(c0 + 1, gbuf1, sem1)
        reduce_into(gbuf0, 0)
        drain(gbuf1, sem1)

        @pl.when(c0 + 2 < SUPER_CHUNKS)
        def _():
          fire(c0 + 2, gbuf0, sem0)

        reduce_into(gbuf1, ROWS_PER_CHUNK)
        epilogue(s * (SUPER_CHUNKS // 2) + p)
        return carry2

      lax.fori_loop(0, SUPER_CHUNKS // 2, pair_body, 0)
      return carry

    lax.fori_loop(0, SUPERS, super_body, 0)
    pltpu.sync_copy(out_v, out_hbm.at[pl.ds(wid * B_PER_W, B_PER_W)])

  return k


_sc_kernel = _make_sc_kernel()


def kernel(x, table, W, b):
  w = W.astype(jnp.float32).reshape(EMBED_DIM, 1)
  q = _q_call(table, w).reshape(VOCAB)
  x2 = x.astype(jnp.int32).reshape(BATCH * HIST)
  bv = jnp.broadcast_to(b.astype(jnp.float32), (EMBED_DIM,))
  out = _sc_kernel(x2, q, bv)
  return out.reshape(BATCH, 1)


# speedup vs baseline: 1.3446x; 1.3446x over previous
import functools

import jax
import jax.numpy as jnp
from jax import lax
from jax.experimental import pallas as pl
from jax.experimental.pallas import tpu as pltpu
from jax.experimental.pallas import tpu_sc as plsc

VOCAB = 1000000
EMBED_DIM = 16
BATCH = 16384
HIST = 200

NC = 2
NS = 16
NW = NC * NS

B_PER_W = BATCH // NW
ROWS_PER_CHUNK = 8
CHUNKS = B_PER_W // ROWS_PER_CHUNK
CHUNK_VALS = ROWS_PER_CHUNK * HIST
VALS_PER_W = B_PER_W * HIST
SUPER_CHUNKS = 16
SUPERS = CHUNKS // SUPER_CHUNKS
VALS_PER_SUPER = SUPER_CHUNKS * CHUNK_VALS
D0, D1 = 96, 104

_RNE_MAGIC = float(2.0 ** 23)


_QROWS = VOCAB // 8
_QBLK = 5000


def _q_body(t_ref, w_ref, o_ref):
  o_ref[...] = jnp.dot(t_ref[...], w_ref[...],
                       preferred_element_type=jnp.float32)


_q_call = pl.pallas_call(
    _q_body,
    grid=(_QROWS // _QBLK,),
    in_specs=[
        pl.BlockSpec((_QBLK, 128), lambda i: (i, 0)),
        pl.BlockSpec((128, 8), lambda i: (0, 0)),
    ],
    out_specs=pl.BlockSpec((_QBLK, 8), lambda i: (i, 0)),
    out_shape=jax.ShapeDtypeStruct((_QROWS, 8), jnp.float32),
)


def _make_sc_kernel():
  mesh = plsc.VectorSubcoreMesh(core_axis_name="c", subcore_axis_name="s")

  @functools.partial(
      pl.kernel,
      mesh=mesh,
      compiler_params=pltpu.CompilerParams(
          needs_layout_passes=False, use_tc_tiling_on_sc=False),
      out_type=jax.ShapeDtypeStruct((BATCH,), jnp.float32),
      scratch_types=[
          pltpu.VMEM_SHARED((VOCAB,), jnp.float32),
          pltpu.VMEM((VALS_PER_SUPER,), jnp.int32),
          pltpu.VMEM((CHUNK_VALS,), jnp.float32),
          pltpu.VMEM((CHUNK_VALS,), jnp.float32),
          pltpu.VMEM((B_PER_W,), jnp.float32),
          pltpu.VMEM((EMBED_DIM,), jnp.float32),
          pltpu.VMEM((EMBED_DIM, 2 * ROWS_PER_CHUNK), jnp.float32),
          pltpu.SemaphoreType.DMA,
          pltpu.SemaphoreType.DMA,
      ],
  )
  def k(x_hbm, q_hbm, b_hbm, out_hbm,
        q_sp, idx_buf, gbuf0, gbuf1, out_v, b_v, macc, sem0, sem1):
    sid = lax.axis_index("s")
    wid = sid * NC + lax.axis_index("c")

    @pl.when(sid == 0)
    def _():
      pltpu.sync_copy(q_hbm, q_sp)

    pltpu.sync_copy(b_hbm, b_v)
    plsc.subcore_barrier()

    bv = b_v[...]
    lane = lax.iota(jnp.int32, 16)
    inv_hist = jnp.float32(1.0 / HIST)
    tail_flat = (jnp.int32(200) * lax.shift_right_logical(lane, 3)
                 + jnp.int32(192) + (lane & 7))
    tadd_rows = lane & 7
    tadd_cols = lax.shift_right_logical(lane, 3)

    def fire(c, gbuf, sem):
      for r in range(ROWS_PER_CHUNK):
        off = pl.multiple_of(c * CHUNK_VALS + r * HIST, 8)
        pltpu.async_copy(
            q_sp.at[idx_buf.at[pl.ds(off, D0)]],
            gbuf.at[pl.ds(r * HIST, D0)],
            sem)
        pltpu.async_copy(
            q_sp.at[idx_buf.at[pl.ds(off + D0, D1)]],
            gbuf.at[pl.ds(r * HIST + D0, D1)],
            sem)

    def drain(gbuf, sem):
      for r in range(ROWS_PER_CHUNK):
        pltpu.make_async_copy(
            q_sp.at[idx_buf.at[pl.ds(0, D0)]],
            gbuf.at[pl.ds(r * HIST, D0)],
            sem).wait()
        pltpu.make_async_copy(
            q_sp.at[idx_buf.at[pl.ds(D0, D1)]],
            gbuf.at[pl.ds(r * HIST + D0, D1)],
            sem).wait()

    def reduce_into(gbuf, col_base):
      for r in range(ROWS_PER_CHUNK):
        base = r * HIST
        accs = [gbuf[pl.ds(base + 16 * t, 16)] for t in range(4)]
        for t in range(4, 12):
          accs[t % 4] = accs[t % 4] + gbuf[pl.ds(base + 16 * t, 16)]
        acc = (accs[0] + accs[1]) + (accs[2] + accs[3])
        plsc.store_scatter(macc, [lane, jnp.full((16,), col_base + r,
                                                 jnp.int32)],
                           acc)
      for g in range(4):
        tails = plsc.load_gather(gbuf, [jnp.int32(400 * g) + tail_flat])
        plsc.addupdate_scatter(
            macc, [tadd_rows,
                   jnp.int32(col_base + 2 * g) + tadd_cols],
            tails)

    def epilogue(i):
      cols = [macc[d, :] for d in range(EMBED_DIM)]
      while len(cols) > 1:
        cols = [cols[i] + cols[i + 1] for i in range(0, len(cols), 2)]
      z = cols[0]
      t = z * inv_hist + bv
      p = 1.0 / (1.0 + jnp.exp(-t))
      y = p * jnp.float32(10000.0)
      y = (y + _RNE_MAGIC) - _RNE_MAGIC
      out_v[pl.ds(i * 16, 16)] = y / jnp.float32(10000.0)

    def super_body(s, carry):
      pltpu.sync_copy(
          x_hbm.at[pl.ds(wid * VALS_PER_W + s * VALS_PER_SUPER,
                         VALS_PER_SUPER)],
          idx_buf)
      fire(0, gbuf0, sem0)

      def pair_body(p, carry2):
        c0 = p * 2
        drain(gbuf0, sem0)
        fire(c0 + 1, gbuf1, sem1)
        reduce_into(gbuf0, 0)
        drain(gbuf1, sem1)

        @pl.when(c0 + 2 < SUPER_CHUNKS)
        def _():
          fire(c0 + 2, gbuf0, sem0)

        reduce_into(gbuf1, ROWS_PER_CHUNK)
        epilogue(s * (SUPER_CHUNKS // 2) + p)
        return carry2

      lax.fori_loop(0, SUPER_CHUNKS // 2, pair_body, 0)
      return carry

    lax.fori_loop(0, SUPERS, super_body, 0)
    pltpu.sync_copy(out_v, out_hbm.at[pl.ds(wid * B_PER_W, B_PER_W)])

  return k


_sc_kernel = _make_sc_kernel()


def kernel(x, table, W, b):
  w = W.astype(jnp.float32).reshape(EMBED_DIM)
  wmat = (jnp.eye(8, dtype=jnp.float32)[:, None, :]
          * w[None, :, None]).reshape(128, 8)
  q = _q_call(table.reshape(_QROWS, 128), wmat).reshape(VOCAB)
  x2 = x.astype(jnp.int32).reshape(BATCH * HIST)
  bv = jnp.broadcast_to(b.astype(jnp.float32), (EMBED_DIM,))
  out = _sc_kernel(x2, q, bv)
  return out.reshape(BATCH, 1)


# speedup vs baseline: 1.3545x; 1.0074x over previous
import functools

import jax
import jax.numpy as jnp
from jax import lax
from jax.experimental import pallas as pl
from jax.experimental.pallas import tpu as pltpu
from jax.experimental.pallas import tpu_sc as plsc

VOCAB = 1000000
EMBED_DIM = 16
BATCH = 16384
HIST = 200

NC = 2
NS = 16
NW = NC * NS

B_PER_W = BATCH // NW
ROWS_PER_CHUNK = 8
CHUNKS = B_PER_W // ROWS_PER_CHUNK
CHUNK_VALS = ROWS_PER_CHUNK * HIST
VALS_PER_W = B_PER_W * HIST
SUPER_CHUNKS = 16
SUPERS = CHUNKS // SUPER_CHUNKS
VALS_PER_SUPER = SUPER_CHUNKS * CHUNK_VALS
D0, D1 = 96, 104

_RNE_MAGIC = float(2.0 ** 23)


_QROWS = VOCAB // 8
_QBLK = 25000


def _q_body(t_ref, w_ref, o_ref):
  o_ref[...] = jnp.dot(t_ref[...], w_ref[...],
                       preferred_element_type=jnp.float32)


_q_call = pl.pallas_call(
    _q_body,
    grid=(_QROWS // _QBLK,),
    in_specs=[
        pl.BlockSpec((_QBLK, 128), lambda i: (i, 0)),
        pl.BlockSpec((128, 8), lambda i: (0, 0)),
    ],
    out_specs=pl.BlockSpec((_QBLK, 8), lambda i: (i, 0)),
    out_shape=jax.ShapeDtypeStruct((_QROWS, 8), jnp.float32),
)


def _make_sc_kernel():
  mesh = plsc.VectorSubcoreMesh(core_axis_name="c", subcore_axis_name="s")

  @functools.partial(
      pl.kernel,
      mesh=mesh,
      compiler_params=pltpu.CompilerParams(
          needs_layout_passes=False, use_tc_tiling_on_sc=False),
      out_type=jax.ShapeDtypeStruct((BATCH,), jnp.float32),
      scratch_types=[
          pltpu.VMEM_SHARED((VOCAB,), jnp.float32),
          pltpu.VMEM((VALS_PER_SUPER,), jnp.int32),
          pltpu.VMEM((CHUNK_VALS,), jnp.float32),
          pltpu.VMEM((CHUNK_VALS,), jnp.float32),
          pltpu.VMEM((B_PER_W,), jnp.float32),
          pltpu.VMEM((EMBED_DIM,), jnp.float32),
          pltpu.VMEM((EMBED_DIM, 2 * ROWS_PER_CHUNK), jnp.float32),
          pltpu.SemaphoreType.DMA,
          pltpu.SemaphoreType.DMA,
      ],
  )
  def k(x_hbm, q_hbm, b_hbm, out_hbm,
        q_sp, idx_buf, gbuf0, gbuf1, out_v, b_v, macc, sem0, sem1):
    sid = lax.axis_index("s")
    wid = sid * NC + lax.axis_index("c")

    @pl.when(sid == 0)
    def _():
      pltpu.sync_copy(q_hbm, q_sp)

    pltpu.sync_copy(b_hbm, b_v)
    plsc.subcore_barrier()

    bv = b_v[...]
    lane = lax.iota(jnp.int32, 16)
    inv_hist = jnp.float32(1.0 / HIST)
    tail_flat = (jnp.int32(200) * lax.shift_right_logical(lane, 3)
                 + jnp.int32(192) + (lane & 7))
    tadd_rows = lane & 7
    tadd_cols = lax.shift_right_logical(lane, 3)

    def fire(c, gbuf, sem):
      for r in range(ROWS_PER_CHUNK):
        off = pl.multiple_of(c * CHUNK_VALS + r * HIST, 8)
        pltpu.async_copy(
            q_sp.at[idx_buf.at[pl.ds(off, D0)]],
            gbuf.at[pl.ds(r * HIST, D0)],
            sem)
        pltpu.async_copy(
            q_sp.at[idx_buf.at[pl.ds(off + D0, D1)]],
            gbuf.at[pl.ds(r * HIST + D0, D1)],
            sem)

    def drain(gbuf, sem):
      for r in range(ROWS_PER_CHUNK):
        pltpu.make_async_copy(
            q_sp.at[idx_buf.at[pl.ds(0, D0)]],
            gbuf.at[pl.ds(r * HIST, D0)],
            sem).wait()
        pltpu.make_async_copy(
            q_sp.at[idx_buf.at[pl.ds(D0, D1)]],
            gbuf.at[pl.ds(r * HIST + D0, D1)],
            sem).wait()

    def reduce_into(gbuf, col_base):
      for r in range(ROWS_PER_CHUNK):
        base = r * HIST
        accs = [gbuf[pl.ds(base + 16 * t, 16)] for t in range(4)]
        for t in range(4, 12):
          accs[t % 4] = accs[t % 4] + gbuf[pl.ds(base + 16 * t, 16)]
        acc = (accs[0] + accs[1]) + (accs[2] + accs[3])
        plsc.store_scatter(macc, [lane, jnp.full((16,), col_base + r,
                                                 jnp.int32)],
                           acc)
      for g in range(4):
        tails = plsc.load_gather(gbuf, [jnp.int32(400 * g) + tail_flat])
        plsc.addupdate_scatter(
            macc, [tadd_rows,
                   jnp.int32(col_base + 2 * g) + tadd_cols],
            tails)

    def epilogue(i):
      cols = [macc[d, :] for d in range(EMBED_DIM)]
      while len(cols) > 1:
        cols = [cols[i] + cols[i + 1] for i in range(0, len(cols), 2)]
      z = cols[0]
      t = z * inv_hist + bv
      p = 1.0 / (1.0 + jnp.exp(-t))
      y = p * jnp.float32(10000.0)
      y = (y + _RNE_MAGIC) - _RNE_MAGIC
      out_v[pl.ds(i * 16, 16)] = y / jnp.float32(10000.0)

    def super_body(s, carry):
      pltpu.sync_copy(
          x_hbm.at[pl.ds(wid * VALS_PER_W + s * VALS_PER_SUPER,
                         VALS_PER_SUPER)],
          idx_buf)
      fire(0, gbuf0, sem0)

      def pair_body(p, carry2):
        c0 = p * 2
        drain(gbuf0, sem0)
        fire(c0 + 1, gbuf1, sem1)
        reduce_into(gbuf0, 0)
        drain(gbuf1, sem1)

        @pl.when(c0 + 2 < SUPER_CHUNKS)
        def _():
          fire(c0 + 2, gbuf0, sem0)

        reduce_into(gbuf1, ROWS_PER_CHUNK)
        epilogue(s * (SUPER_CHUNKS // 2) + p)
        return carry2

      lax.fori_loop(0, SUPER_CHUNKS // 2, pair_body, 0)
      return carry

    lax.fori_loop(0, SUPERS, super_body, 0)
    pltpu.sync_copy(out_v, out_hbm.at[pl.ds(wid * B_PER_W, B_PER_W)])

  return k


_sc_kernel = _make_sc_kernel()


def kernel(x, table, W, b):
  w = W.astype(jnp.float32).reshape(EMBED_DIM)
  wmat = (jnp.eye(8, dtype=jnp.float32)[:, None, :]
          * w[None, :, None]).reshape(128, 8)
  q = _q_call(table.reshape(_QROWS, 128), wmat).reshape(VOCAB)
  x2 = x.astype(jnp.int32).reshape(BATCH * HIST)
  bv = jnp.broadcast_to(b.astype(jnp.float32), (EMBED_DIM,))
  out = _sc_kernel(x2, q, bv)
  return out.reshape(BATCH, 1)
